# Initial kernel scaffold; baseline (speedup 1.0000x reference)
import jax
import jax.numpy as jnp
from jax.experimental import pallas as pl


def kernel(x, h, edge_index, msg_W1, msg_b1, msg_W2, msg_b2, coord_W1, coord_b1, coord_W2, coord_b2, node_W1, node_b1, node_W2, node_b2):
    raise NotImplementedError("write your pallas kernel here")


# speedup vs baseline: 4.0061x; 4.0061x over previous
import functools

import jax
import jax.numpy as jnp
from jax import lax
from jax.experimental import pallas as pl
from jax.experimental.pallas import tpu as pltpu
from jax.experimental.pallas import tpu_sc as plsc

N = 10000
E = 320000
EMB = 128
HID = 64
D = 80
CHUNK = 128
NCHUNKS = E // CHUNK
NC = 2
NS = 16
NW = NC * NS
KPT = -(-NCHUNKS // NW)
ROWS_PT = N // NS
ZROWS = 125

_f32 = jnp.float32


def _tables_body(h_ref, x_ref, w1a_ref, w1b_ref, b1_ref, ts_ref, td_ref):
    h = h_ref[...]
    nb = h.shape[0]
    pad = jnp.zeros((nb, D - HID - 3), _f32)
    a = jnp.dot(h, w1a_ref[...], preferred_element_type=_f32) + b1_ref[...]
    ts_ref[...] = jnp.concatenate([a, x_ref[...], pad], axis=1)
    b = jnp.dot(h, w1b_ref[...], preferred_element_type=_f32)
    td_ref[...] = jnp.concatenate([b, -x_ref[...], pad], axis=1)


def _build_tables(h, x, w1a, w1b, b1):
    nb = 2000
    grid = N // nb
    return pl.pallas_call(
        _tables_body,
        grid=(grid,),
        in_specs=[
            pl.BlockSpec((nb, EMB), lambda i: (i, 0)),
            pl.BlockSpec((nb, 3), lambda i: (i, 0)),
            pl.BlockSpec((EMB, HID), lambda i: (0, 0)),
            pl.BlockSpec((EMB, HID), lambda i: (0, 0)),
            pl.BlockSpec((1, HID), lambda i: (0, 0)),
        ],
        out_specs=[
            pl.BlockSpec((nb, D), lambda i: (i, 0)),
            pl.BlockSpec((nb, D), lambda i: (i, 0)),
        ],
        out_shape=[
            jax.ShapeDtypeStruct((N, D), _f32),
            jax.ShapeDtypeStruct((N, D), _f32),
        ],
    )(h, x, w1a, w1b, b1)


def _gather_body(tsrc, tdst, row, col, z_out, idx_r, idx_c, zbuf, sem1, sem2):
    cid = lax.axis_index("c")
    sid = lax.axis_index("s")
    wid = sid * NC + cid

    def body(k, carry):
        c = wid + k * NW

        @pl.when(c < NCHUNKS)
        def _():
            base = c * CHUNK
            pltpu.sync_copy(row.at[pl.ds(base, CHUNK)], idx_r)
            pltpu.sync_copy(col.at[pl.ds(base, CHUNK)], idx_c)
            pltpu.async_copy(tsrc.at[idx_r], zbuf, sem1).wait()
            pltpu.async_copy(tdst.at[idx_c], zbuf, sem2, add=True).wait()
            pltpu.sync_copy(zbuf, z_out.at[pl.ds(base, CHUNK)])

        return carry

    lax.fori_loop(0, KPT, body, 0)


def _gather_edges(tsrc, tdst, row, col):
    mesh = plsc.VectorSubcoreMesh(
        core_axis_name="c", subcore_axis_name="s", num_cores=NC, num_subcores=NS
    )
    return pl.kernel(
        _gather_body,
        out_type=jax.ShapeDtypeStruct((E, D), _f32),
        mesh=mesh,
        compiler_params=pltpu.CompilerParams(use_tc_tiling_on_sc=False),
        scratch_types=[
            pltpu.VMEM((CHUNK,), jnp.int32),
            pltpu.VMEM((CHUNK,), jnp.int32),
            pltpu.VMEM((CHUNK, D), _f32),
            pltpu.SemaphoreType.DMA,
            pltpu.SemaphoreType.DMA,
        ],
    )(tsrc, tdst, row, col)


def _edge_mlp_body(z_ref, s1_ref, sq_ref, s2_ref, srad_ref, w2_ref, b2_ref,
                   cw1_ref, cb1_ref, cw2_ref, cb2_ref, u_ref):
    z = z_ref[...]
    nb = z.shape[0]
    zz = z * z
    rad = jnp.dot(zz, srad_ref[...], preferred_element_type=_f32)
    z1 = jax.nn.relu(
        jnp.dot(z, s1_ref[...], preferred_element_type=_f32)
        + jnp.dot(zz, sq_ref[...], preferred_element_type=_f32)
    )
    msgs = jax.nn.relu(
        jnp.dot(z1, w2_ref[...], preferred_element_type=_f32) + b2_ref[...]
    )
    t = jax.nn.relu(
        jnp.dot(msgs, cw1_ref[...], preferred_element_type=_f32) + cb1_ref[...]
    )
    cc = jnp.dot(t, cw2_ref[...], preferred_element_type=_f32) + cb2_ref[...]
    scale = cc * lax.rsqrt(rad + 1e-8)
    cu = jnp.dot(z, s2_ref[...], preferred_element_type=_f32) * scale
    pad = jnp.zeros((nb, D - HID - 4), _f32)
    u_ref[...] = jnp.concatenate([msgs, cu, pad], axis=1)


def _edge_mlp(z, s1, sq, s2, srad, w2, b2, cw1, cb1, cw2, cb2):
    eb = 1280
    grid = E // eb
    full = lambda r, c: pl.BlockSpec((r, c), lambda i: (0, 0))
    return pl.pallas_call(
        _edge_mlp_body,
        grid=(grid,),
        in_specs=[
            pl.BlockSpec((eb, D), lambda i: (i, 0)),
            full(D, HID), full(D, HID), full(D, 4), full(D, 1),
            full(HID, HID), full(1, HID),
            full(HID, HID), full(1, HID), full(HID, 1), full(1, 1),
        ],
        out_specs=pl.BlockSpec((eb, D), lambda i: (i, 0)),
        out_shape=jax.ShapeDtypeStruct((E, D), _f32),
    )(z, s1, sq, s2, srad, w2, b2, cw1, cb1, cw2, cb2)


def _scatter_body(u, row, zeros_hbm, p_out, ubuf, idxb, zbuf, acc):
    cid = lax.axis_index("c")
    sid = lax.axis_index("s")
    wid = sid * NC + cid
    row0 = sid * ROWS_PT

    pltpu.sync_copy(zeros_hbm.at[pl.ds(0, ZROWS)], zbuf)
    for j in range(ROWS_PT // ZROWS):
        pltpu.sync_copy(zbuf, acc.at[pl.ds(row0 + j * ZROWS, ZROWS)])
    plsc.subcore_barrier()

    def body(k, carry):
        c = wid + k * NW

        @pl.when(c < NCHUNKS)
        def _():
            base = c * CHUNK
            pltpu.sync_copy(row.at[pl.ds(base, CHUNK)], idxb)
            pltpu.sync_copy(u.at[pl.ds(base, CHUNK)], ubuf)
            pltpu.sync_copy(ubuf, acc.at[idxb], add=True)

        return carry

    lax.fori_loop(0, KPT, body, 0)
    plsc.subcore_barrier()

    for j in range(ROWS_PT // ZROWS):
        r = row0 + j * ZROWS
        pltpu.sync_copy(acc.at[pl.ds(r, ZROWS)], zbuf)
        pltpu.sync_copy(zbuf, p_out.at[cid].at[pl.ds(r, ZROWS)])


def _scatter_edges(u, row, zeros_hbm):
    mesh = plsc.VectorSubcoreMesh(
        core_axis_name="c", subcore_axis_name="s", num_cores=NC, num_subcores=NS
    )
    return pl.kernel(
        _scatter_body,
        out_type=jax.ShapeDtypeStruct((NC, N, D), _f32),
        mesh=mesh,
        compiler_params=pltpu.CompilerParams(use_tc_tiling_on_sc=False),
        scratch_types=[
            pltpu.VMEM((CHUNK, D), _f32),
            pltpu.VMEM((CHUNK,), jnp.int32),
            pltpu.VMEM((ZROWS, D), _f32),
            pltpu.VMEM_SHARED((N, D), _f32),
        ],
    )(u, row, zeros_hbm)


def _node_mlp_body(p_ref, h_ref, x_ref, w1f_ref, w1h_ref, b1_ref, w2_ref,
                   b2_ref, sx_ref, xo_ref, ho_ref):
    g = p_ref[0] + p_ref[1]
    h = h_ref[...]
    t = jax.nn.relu(
        jnp.dot(g, w1f_ref[...], preferred_element_type=_f32)
        + jnp.dot(h, w1h_ref[...], preferred_element_type=_f32)
        + b1_ref[...]
    )
    ho_ref[...] = h + jnp.dot(t, w2_ref[...], preferred_element_type=_f32) \
        + b2_ref[...]
    xo_ref[...] = x_ref[...] + jnp.dot(g, sx_ref[...],
                                       preferred_element_type=_f32)


def _node_mlp(p, h, x, w1f_ext, w1h, b1, w2, b2, sx):
    nb = 2000
    grid = N // nb
    full = lambda r, c: pl.BlockSpec((r, c), lambda i: (0, 0))
    return pl.pallas_call(
        _node_mlp_body,
        grid=(grid,),
        in_specs=[
            pl.BlockSpec((NC, nb, D), lambda i: (0, i, 0)),
            pl.BlockSpec((nb, EMB), lambda i: (i, 0)),
            pl.BlockSpec((nb, 3), lambda i: (i, 0)),
            full(D, HID), full(EMB, HID), full(1, HID),
            full(HID, EMB), full(1, EMB), full(D, 3),
        ],
        out_specs=[
            pl.BlockSpec((nb, 3), lambda i: (i, 0)),
            pl.BlockSpec((nb, EMB), lambda i: (i, 0)),
        ],
        out_shape=[
            jax.ShapeDtypeStruct((N, 3), _f32),
            jax.ShapeDtypeStruct((N, EMB), _f32),
        ],
    )(p, h, x, w1f_ext, w1h, b1, w2, b2, sx)


def kernel(x, h, edge_index, msg_W1, msg_b1, msg_W2, msg_b2, coord_W1,
           coord_b1, coord_W2, coord_b2, node_W1, node_b1, node_W2, node_b2):
    ei = edge_index.astype(jnp.int32)
    row = ei[0]
    col = ei[1]

    w1a = msg_W1[:EMB]
    w1b = msg_W1[EMB:2 * EMB]
    w1c = msg_W1[2 * EMB]

    tsrc, tdst = _build_tables(h, x, w1a, w1b, msg_b1.reshape(1, HID))
    z = _gather_edges(tsrc, tdst, row, col)

    eye = jnp.eye(D, dtype=_f32)
    s1 = eye[:, :HID]
    sq = jnp.zeros((D, HID), _f32).at[HID:HID + 3].set(
        jnp.broadcast_to(w1c, (3, HID)))
    s2 = eye[:, HID:HID + 4]
    srad = jnp.zeros((D, 1), _f32).at[HID:HID + 3].set(1.0)

    u = _edge_mlp(z, s1, sq, s2, srad, msg_W2, msg_b2.reshape(1, HID),
                  coord_W1, coord_b1.reshape(1, HID), coord_W2,
                  coord_b2.reshape(1, 1))

    p = _scatter_edges(u, row, jnp.zeros((ZROWS, D), _f32))

    w1f_ext = jnp.zeros((D, HID), _f32).at[:HID].set(node_W1[:HID])
    sx = jnp.zeros((D, 3), _f32).at[HID:HID + 3].set(jnp.eye(3, dtype=_f32))

    x_new, h_new = _node_mlp(p, h, x, w1f_ext, node_W1[HID:],
                             node_b1.reshape(1, HID), node_W2,
                             node_b2.reshape(1, EMB), sx)
    return (x_new, h_new)
